# baseline (speedup 1.0000x reference)
import functools

import jax
import jax.numpy as jnp
from jax import lax
from jax.experimental import pallas as pl
from jax.experimental.pallas import tpu as pltpu
from jax.experimental.pallas import tpu_sc as plsc

_B = 4096
_L = 200
_D = 64
_HD = 32
_VOCAB = 1000000
_CHUNK = 100

_VB = 4096
_NQ = 62
_S4 = _VB * _NQ


def _make_sc_pool(num_cores: int, num_subcores: int, d: int):
    nw = num_cores * num_subcores
    rows_per_w = _B // nw
    chunks_per_w = rows_per_w * 2
    nvec = d // 16

    mesh = plsc.VectorSubcoreMesh(core_axis_name="c", subcore_axis_name="s")

    @functools.partial(
        pl.kernel,
        mesh=mesh,
        out_type=jax.ShapeDtypeStruct((_B, d), jnp.float32),
        compiler_params=pltpu.CompilerParams(use_tc_tiling_on_sc=False),
        scratch_types=[
            pltpu.VMEM((chunks_per_w, _CHUNK), jnp.int32),
            pltpu.VMEM((_CHUNK, d), jnp.float32),
            pltpu.VMEM((_CHUNK, d), jnp.float32),
            pltpu.VMEM((rows_per_w, d), jnp.float32),
            pltpu.SemaphoreType.DMA,
        ],
    )
    def sc_pool(idx_hbm, table_hbm, out_hbm, idx_v, buf_a, buf_b, out_v, sem):
        wid = lax.axis_index("s") * num_cores + lax.axis_index("c")
        pltpu.sync_copy(idx_hbm.at[pl.ds(wid * chunks_per_w, chunks_per_w)],
                        idx_v)

        def body(i, carry):
            cp_a = pltpu.async_copy(table_hbm.at[idx_v.at[2 * i]], buf_a, sem)
            cp_b = pltpu.async_copy(table_hbm.at[idx_v.at[2 * i + 1]], buf_b,
                                    sem)
            cp_a.wait()
            cp_b.wait()

            def accum(j, acc):
                return tuple(
                    acc[k]
                    + buf_a[j, pl.ds(16 * k, 16)]
                    + buf_b[j, pl.ds(16 * k, 16)]
                    for k in range(nvec)
                )

            zero = jnp.zeros((16,), jnp.float32)
            acc = lax.fori_loop(0, _CHUNK, accum, (zero,) * nvec)
            for k in range(nvec):
                out_v[i, pl.ds(16 * k, 16)] = acc[k]
            return carry

        lax.fori_loop(0, rows_per_w, body, 0)
        pltpu.sync_copy(out_v, out_hbm.at[pl.ds(wid * rows_per_w, rows_per_w)])

    return sc_pool


def _transpose_body(a_ref, b_ref, c_ref, d_ref, o_ref):
    o_ref[:, 0:_HD] = a_ref[...].T
    o_ref[:, _HD : 2 * _HD] = b_ref[...].T
    o_ref[:, 2 * _HD : 3 * _HD] = c_ref[...].T
    o_ref[:, 3 * _HD : 4 * _HD] = d_ref[...].T


def _transpose_half(tableT, h: int):
    def _spec(q):
        return pl.BlockSpec(
            (_HD, _VB),
            lambda i, q=q: (h, jnp.minimum(q * _NQ + i, _VOCAB // _VB)),
        )

    return pl.pallas_call(
        _transpose_body,
        grid=(_NQ,),
        in_specs=[_spec(0), _spec(1), _spec(2), _spec(3)],
        out_specs=pl.BlockSpec((_VB, 128), lambda i: (i, 0)),
        out_shape=jax.ShapeDtypeStruct((_S4, 128), jnp.float32),
    )(tableT, tableT, tableT, tableT)


def _mlp_body(x_ref, w1_ref, b1_ref, w2_ref, b2_ref, w3_ref, b3_ref, o_ref):
    x = x_ref[...]
    h = jnp.tanh(jnp.dot(x, w1_ref[...],
                         preferred_element_type=jnp.float32) + b1_ref[...])
    h = jnp.tanh(jnp.dot(h, w2_ref[...],
                         preferred_element_type=jnp.float32) + b2_ref[...])
    o_ref[...] = jnp.dot(h, w3_ref[...],
                         preferred_element_type=jnp.float32) + b3_ref[...]


def kernel(inputs, table, W1, b1, W2, b2, W3, b3):
    info = plsc.get_sparse_core_info()
    sc_pool = _make_sc_pool(info.num_cores, info.num_subcores, _HD)

    q = inputs // _S4
    ridx = 4 * inputs - q * (4 * _S4 - 1)
    idx2 = ridx.reshape(_B * 2, _CHUNK)

    tableT = table.T
    halves = []
    for h in range(2):
        tab_lin = _transpose_half(tableT, h).reshape(4 * _S4, _HD)
        halves.append(sc_pool(idx2, tab_lin))
    x = jnp.concatenate(halves, axis=1)

    h1 = W1.shape[1]
    w1p = jnp.pad(W1, ((0, 0), (0, 128 - h1)))
    b1p = jnp.pad(b1, (0, 128 - h1)).reshape(1, 128)
    w2p = jnp.pad(W2, ((0, 128 - h1), (0, 128 - h1)))
    b2p = jnp.pad(b2, (0, 128 - h1)).reshape(1, 128)
    w3p = jnp.pad(W3, ((0, 128 - h1), (0, 128 - W3.shape[1])))
    b3p = jnp.pad(b3, (0, 128 - W3.shape[1])).reshape(1, 128)

    logits_pad = pl.pallas_call(
        _mlp_body,
        out_shape=jax.ShapeDtypeStruct((_B, 128), jnp.float32),
    )(x, w1p, b1p, w2p, b2p, w3p, b3p)
    return logits_pad[:, : W3.shape[1]]


# speedup vs baseline: 1.6931x; 1.6931x over previous
import functools

import jax
import jax.numpy as jnp
from jax import lax
from jax.experimental import pallas as pl
from jax.experimental.pallas import tpu as pltpu
from jax.experimental.pallas import tpu_sc as plsc

_B = 4096
_L = 200
_D = 64
_VOCAB = 1000000
_CHUNK = 100


def _make_sc_pool(num_cores: int, num_subcores: int):
    nw = num_cores * num_subcores
    rows_per_w = _B // nw
    chunks_per_w = rows_per_w * 2

    mesh = plsc.VectorSubcoreMesh(core_axis_name="c", subcore_axis_name="s")

    @functools.partial(
        pl.kernel,
        mesh=mesh,
        out_type=jax.ShapeDtypeStruct((_B, _D), jnp.float32),
        compiler_params=pltpu.CompilerParams(use_tc_tiling_on_sc=False),
        scratch_types=[
            pltpu.VMEM((chunks_per_w, _CHUNK), jnp.int32),
            pltpu.VMEM((2, _CHUNK, _D), jnp.float32),
            pltpu.VMEM((2, _CHUNK, _D), jnp.float32),
            pltpu.VMEM((rows_per_w, _D), jnp.float32),
            pltpu.SemaphoreType.DMA,
            pltpu.SemaphoreType.DMA,
        ],
    )
    def sc_pool(idx_hbm, table_hbm, out_hbm, idx_v, buf_a, buf_b, out_v,
                sem0, sem1):
        wid = lax.axis_index("s") * num_cores + lax.axis_index("c")
        pltpu.sync_copy(idx_hbm.at[pl.ds(wid * chunks_per_w, chunks_per_w)],
                        idx_v)
        sems = (sem0, sem1)

        def start(i, s):
            pltpu.async_copy(table_hbm.at[idx_v.at[2 * i]], buf_a.at[s],
                             sems[s])
            pltpu.async_copy(table_hbm.at[idx_v.at[2 * i + 1]], buf_b.at[s],
                             sems[s])

        def drain(s):
            pltpu.make_async_copy(table_hbm.at[idx_v.at[0]], buf_a.at[s],
                                  sems[s]).wait()
            pltpu.make_async_copy(table_hbm.at[idx_v.at[0]], buf_b.at[s],
                                  sems[s]).wait()

        def consume(i, s):
            def accum(j, acc):
                a0, a1, a2, a3 = acc
                a0 = a0 + buf_a[s, j, pl.ds(0, 16)] + buf_b[s, j, pl.ds(0, 16)]
                a1 = (a1 + buf_a[s, j, pl.ds(16, 16)]
                      + buf_b[s, j, pl.ds(16, 16)])
                a2 = (a2 + buf_a[s, j, pl.ds(32, 16)]
                      + buf_b[s, j, pl.ds(32, 16)])
                a3 = (a3 + buf_a[s, j, pl.ds(48, 16)]
                      + buf_b[s, j, pl.ds(48, 16)])
                return (a0, a1, a2, a3)

            zero = jnp.zeros((16,), jnp.float32)
            a0, a1, a2, a3 = lax.fori_loop(0, _CHUNK, accum,
                                           (zero, zero, zero, zero))
            out_v[i, pl.ds(0, 16)] = a0
            out_v[i, pl.ds(16, 16)] = a1
            out_v[i, pl.ds(32, 16)] = a2
            out_v[i, pl.ds(48, 16)] = a3

        start(0, 0)

        def body(g, carry):
            i0 = 2 * g
            start(i0 + 1, 1)
            drain(0)
            consume(i0, 0)
            start(i0 + 2, 0)
            drain(1)
            consume(i0 + 1, 1)
            return carry

        lax.fori_loop(0, rows_per_w // 2 - 1, body, 0)
        i0 = rows_per_w - 2
        start(i0 + 1, 1)
        drain(0)
        consume(i0, 0)
        drain(1)
        consume(i0 + 1, 1)
        pltpu.sync_copy(out_v, out_hbm.at[pl.ds(wid * rows_per_w, rows_per_w)])

    return sc_pool


_VB = 4096
_NTB = 123
_SPLIT = _VB * _NTB


def _transpose_body(a_ref, b_ref, o_ref):
    o_ref[:, 0:_D] = a_ref[...].T
    o_ref[:, _D : 2 * _D] = b_ref[...].T


def _transpose_table(tableT):
    return pl.pallas_call(
        _transpose_body,
        grid=(_NTB,),
        in_specs=[
            pl.BlockSpec((_D, _VB), lambda i: (0, i)),
            pl.BlockSpec(
                (_D, _VB), lambda i: (0, jnp.minimum(_NTB + i, _VOCAB // _VB))
            ),
        ],
        out_specs=pl.BlockSpec((_VB, 128), lambda i: (i, 0)),
        out_shape=jax.ShapeDtypeStruct((_SPLIT, 128), jnp.float32),
    )(tableT, tableT)


def _mlp_body(x_ref, w1_ref, b1_ref, w2_ref, b2_ref, w3_ref, b3_ref, o_ref):
    x = x_ref[...]
    h = jnp.tanh(jnp.dot(x, w1_ref[...],
                         preferred_element_type=jnp.float32) + b1_ref[...])
    h = jnp.tanh(jnp.dot(h, w2_ref[...],
                         preferred_element_type=jnp.float32) + b2_ref[...])
    o_ref[...] = jnp.dot(h, w3_ref[...],
                         preferred_element_type=jnp.float32) + b3_ref[...]


def kernel(inputs, table, W1, b1, W2, b2, W3, b3):
    info = plsc.get_sparse_core_info()
    sc_pool = _make_sc_pool(info.num_cores, info.num_subcores)

    ridx = 2 * inputs - jnp.where(inputs >= _SPLIT, 2 * _SPLIT - 1, 0)
    idx2 = ridx.reshape(_B * 2, _CHUNK)
    tab_lin = _transpose_table(table.T).reshape(2 * _SPLIT, _D)
    x = sc_pool(idx2, tab_lin)

    h1 = W1.shape[1]
    w1p = jnp.pad(W1, ((0, 0), (0, 128 - h1)))
    b1p = jnp.pad(b1, (0, 128 - h1)).reshape(1, 128)
    w2p = jnp.pad(W2, ((0, 128 - h1), (0, 128 - h1)))
    b2p = jnp.pad(b2, (0, 128 - h1)).reshape(1, 128)
    w3p = jnp.pad(W3, ((0, 128 - h1), (0, 128 - W3.shape[1])))
    b3p = jnp.pad(b3, (0, 128 - W3.shape[1])).reshape(1, 128)

    logits_pad = pl.pallas_call(
        _mlp_body,
        out_shape=jax.ShapeDtypeStruct((_B, 128), jnp.float32),
    )(x, w1p, b1p, w2p, b2p, w3p, b3p)
    return logits_pad[:, : W3.shape[1]]


# speedup vs baseline: 1.8393x; 1.0864x over previous
import functools

import jax
import jax.numpy as jnp
from jax import lax
from jax.experimental import pallas as pl
from jax.experimental.pallas import tpu as pltpu
from jax.experimental.pallas import tpu_sc as plsc

_B = 4096
_L = 200
_D = 64
_VOCAB = 1000000
_CHUNK = 100


def _make_sc_pool(num_cores: int, num_subcores: int):
    nw = num_cores * num_subcores
    rows_per_w = _B // nw
    chunks_per_w = rows_per_w * 2

    mesh = plsc.VectorSubcoreMesh(core_axis_name="c", subcore_axis_name="s")

    @functools.partial(
        pl.kernel,
        mesh=mesh,
        out_type=jax.ShapeDtypeStruct((_B, _D), jnp.float32),
        compiler_params=pltpu.CompilerParams(use_tc_tiling_on_sc=False),
        scratch_types=[
            pltpu.VMEM((chunks_per_w, _CHUNK), jnp.int32),
            pltpu.VMEM((2, _CHUNK, _D), jnp.float32),
            pltpu.VMEM((2, _CHUNK, _D), jnp.float32),
            pltpu.VMEM((rows_per_w, _D), jnp.float32),
            pltpu.SemaphoreType.DMA,
            pltpu.SemaphoreType.DMA,
        ],
    )
    def sc_pool(idx_hbm, table_hbm, out_hbm, idx_v, buf_a, buf_b, out_v,
                sem0, sem1):
        wid = lax.axis_index("s") * num_cores + lax.axis_index("c")
        pltpu.sync_copy(idx_hbm.at[pl.ds(wid * chunks_per_w, chunks_per_w)],
                        idx_v)
        sems = (sem0, sem1)

        def start(i, s):
            pltpu.async_copy(table_hbm.at[idx_v.at[2 * i]], buf_a.at[s],
                             sems[s])
            pltpu.async_copy(table_hbm.at[idx_v.at[2 * i + 1]], buf_b.at[s],
                             sems[s])

        def drain(s):
            pltpu.make_async_copy(table_hbm.at[idx_v.at[0]], buf_a.at[s],
                                  sems[s]).wait()
            pltpu.make_async_copy(table_hbm.at[idx_v.at[0]], buf_b.at[s],
                                  sems[s]).wait()

        def consume(i, s):
            def accum(j, acc):
                a0, a1, a2, a3 = acc
                a0 = a0 + buf_a[s, j, pl.ds(0, 16)] + buf_b[s, j, pl.ds(0, 16)]
                a1 = (a1 + buf_a[s, j, pl.ds(16, 16)]
                      + buf_b[s, j, pl.ds(16, 16)])
                a2 = (a2 + buf_a[s, j, pl.ds(32, 16)]
                      + buf_b[s, j, pl.ds(32, 16)])
                a3 = (a3 + buf_a[s, j, pl.ds(48, 16)]
                      + buf_b[s, j, pl.ds(48, 16)])
                return (a0, a1, a2, a3)

            zero = jnp.zeros((16,), jnp.float32)
            a0, a1, a2, a3 = lax.fori_loop(0, _CHUNK, accum,
                                           (zero, zero, zero, zero))
            out_v[i, pl.ds(0, 16)] = a0
            out_v[i, pl.ds(16, 16)] = a1
            out_v[i, pl.ds(32, 16)] = a2
            out_v[i, pl.ds(48, 16)] = a3

        start(0, 0)

        def body(g, carry):
            i0 = 2 * g
            start(i0 + 1, 1)
            drain(0)
            consume(i0, 0)
            start(i0 + 2, 0)
            drain(1)
            consume(i0 + 1, 1)
            return carry

        lax.fori_loop(0, rows_per_w // 2 - 1, body, 0)
        i0 = rows_per_w - 2
        start(i0 + 1, 1)
        drain(0)
        consume(i0, 0)
        drain(1)
        consume(i0 + 1, 1)
        pltpu.sync_copy(out_v, out_hbm.at[pl.ds(wid * rows_per_w, rows_per_w)])

    return sc_pool


_VB = 8192
_NTB = 62
_SPLIT = _VB * _NTB


def _transpose_body(a_ref, b_ref, o_ref):
    o_ref[:, 0:_D] = a_ref[...].T
    o_ref[:, _D : 2 * _D] = b_ref[...].T


def _transpose_table(tableT):
    return pl.pallas_call(
        _transpose_body,
        grid=(_NTB,),
        in_specs=[
            pl.BlockSpec((_D, _VB), lambda i: (0, i)),
            pl.BlockSpec(
                (_D, _VB), lambda i: (0, jnp.minimum(_NTB + i, _VOCAB // _VB))
            ),
        ],
        out_specs=pl.BlockSpec((_VB, 128), lambda i: (i, 0)),
        out_shape=jax.ShapeDtypeStruct((_SPLIT, 128), jnp.float32),
    )(tableT, tableT)


def _mlp_body(x_ref, w1_ref, b1_ref, w2_ref, b2_ref, w3_ref, b3_ref, o_ref):
    x = x_ref[...]
    h = jnp.tanh(jnp.dot(x, w1_ref[...],
                         preferred_element_type=jnp.float32) + b1_ref[...])
    h = jnp.tanh(jnp.dot(h, w2_ref[...],
                         preferred_element_type=jnp.float32) + b2_ref[...])
    o_ref[...] = jnp.dot(h, w3_ref[...],
                         preferred_element_type=jnp.float32) + b3_ref[...]


def kernel(inputs, table, W1, b1, W2, b2, W3, b3):
    info = plsc.get_sparse_core_info()
    sc_pool = _make_sc_pool(info.num_cores, info.num_subcores)

    ridx = 2 * inputs - jnp.where(inputs >= _SPLIT, 2 * _SPLIT - 1, 0)
    idx2 = ridx.reshape(_B * 2, _CHUNK)
    tab_lin = _transpose_table(table.T).reshape(2 * _SPLIT, _D)
    x = sc_pool(idx2, tab_lin)

    h1 = W1.shape[1]
    w1p = jnp.pad(W1, ((0, 0), (0, 128 - h1)))
    b1p = jnp.pad(b1, (0, 128 - h1)).reshape(1, 128)
    w2p = jnp.pad(W2, ((0, 128 - h1), (0, 128 - h1)))
    b2p = jnp.pad(b2, (0, 128 - h1)).reshape(1, 128)
    w3p = jnp.pad(W3, ((0, 128 - h1), (0, 128 - W3.shape[1])))
    b3p = jnp.pad(b3, (0, 128 - W3.shape[1])).reshape(1, 128)

    logits_pad = pl.pallas_call(
        _mlp_body,
        out_shape=jax.ShapeDtypeStruct((_B, 128), jnp.float32),
    )(x, w1p, b1p, w2p, b2p, w3p, b3p)
    return logits_pad[:, : W3.shape[1]]


# speedup vs baseline: 1.9093x; 1.0380x over previous
import functools

import jax
import jax.numpy as jnp
from jax import lax
from jax.experimental import pallas as pl
from jax.experimental.pallas import tpu as pltpu
from jax.experimental.pallas import tpu_sc as plsc

_B = 4096
_L = 200
_D = 64
_VOCAB = 1000000
_CHUNK = 100


def _make_sc_pool(num_cores: int, num_subcores: int):
    nw = num_cores * num_subcores
    rows_per_w = _B // nw
    chunks_per_w = rows_per_w * 2

    mesh = plsc.VectorSubcoreMesh(core_axis_name="c", subcore_axis_name="s")

    @functools.partial(
        pl.kernel,
        mesh=mesh,
        out_type=jax.ShapeDtypeStruct((_B, _D), jnp.float32),
        compiler_params=pltpu.CompilerParams(use_tc_tiling_on_sc=False),
        scratch_types=[
            pltpu.VMEM((chunks_per_w, _CHUNK), jnp.int32),
            pltpu.VMEM((2, _CHUNK, _D), jnp.float32),
            pltpu.VMEM((2, _CHUNK, _D), jnp.float32),
            pltpu.VMEM((rows_per_w, _D), jnp.float32),
            pltpu.SemaphoreType.DMA,
            pltpu.SemaphoreType.DMA,
        ],
    )
    def sc_pool(idx_hbm, table_hbm, out_hbm, idx_v, buf_a, buf_b, out_v,
                sem0, sem1):
        wid = lax.axis_index("s") * num_cores + lax.axis_index("c")
        pltpu.sync_copy(idx_hbm.at[pl.ds(wid * chunks_per_w, chunks_per_w)],
                        idx_v)
        sems = (sem0, sem1)

        def start(i, s):
            pltpu.async_copy(table_hbm.at[idx_v.at[2 * i]], buf_a.at[s],
                             sems[s])
            pltpu.async_copy(table_hbm.at[idx_v.at[2 * i + 1]], buf_b.at[s],
                             sems[s])

        def drain(s):
            pltpu.make_async_copy(table_hbm.at[idx_v.at[0]], buf_a.at[s],
                                  sems[s]).wait()
            pltpu.make_async_copy(table_hbm.at[idx_v.at[0]], buf_b.at[s],
                                  sems[s]).wait()

        def consume(i, s):
            def accum(j, acc):
                a0, a1, a2, a3 = acc
                a0 = a0 + buf_a[s, j, pl.ds(0, 16)] + buf_b[s, j, pl.ds(0, 16)]
                a1 = (a1 + buf_a[s, j, pl.ds(16, 16)]
                      + buf_b[s, j, pl.ds(16, 16)])
                a2 = (a2 + buf_a[s, j, pl.ds(32, 16)]
                      + buf_b[s, j, pl.ds(32, 16)])
                a3 = (a3 + buf_a[s, j, pl.ds(48, 16)]
                      + buf_b[s, j, pl.ds(48, 16)])
                return (a0, a1, a2, a3)

            zero = jnp.zeros((16,), jnp.float32)
            a0, a1, a2, a3 = lax.fori_loop(0, _CHUNK, accum,
                                           (zero, zero, zero, zero))
            out_v[i, pl.ds(0, 16)] = a0
            out_v[i, pl.ds(16, 16)] = a1
            out_v[i, pl.ds(32, 16)] = a2
            out_v[i, pl.ds(48, 16)] = a3

        start(0, 0)

        def body(g, carry):
            i0 = 2 * g
            start(i0 + 1, 1)
            drain(0)
            consume(i0, 0)
            start(i0 + 2, 0)
            drain(1)
            consume(i0 + 1, 1)
            return carry

        lax.fori_loop(0, rows_per_w // 2 - 1, body, 0)
        i0 = rows_per_w - 2
        start(i0 + 1, 1)
        drain(0)
        consume(i0, 0)
        drain(1)
        consume(i0 + 1, 1)
        pltpu.sync_copy(out_v, out_hbm.at[pl.ds(wid * rows_per_w, rows_per_w)])

    return sc_pool


_VB = 16384
_NTB = 31
_SPLIT = _VB * _NTB


def _transpose_body(a_ref, b_ref, o_ref):
    o_ref[:, 0:_D] = a_ref[...].T
    o_ref[:, _D : 2 * _D] = b_ref[...].T


def _transpose_table(tableT):
    return pl.pallas_call(
        _transpose_body,
        grid=(_NTB,),
        in_specs=[
            pl.BlockSpec((_D, _VB), lambda i: (0, i)),
            pl.BlockSpec(
                (_D, _VB), lambda i: (0, jnp.minimum(_NTB + i, _VOCAB // _VB))
            ),
        ],
        out_specs=pl.BlockSpec((_VB, 128), lambda i: (i, 0)),
        out_shape=jax.ShapeDtypeStruct((_SPLIT, 128), jnp.float32),
    )(tableT, tableT)


def _mlp_body(x_ref, w1_ref, b1_ref, w2_ref, b2_ref, w3_ref, b3_ref, o_ref):
    x = x_ref[...]
    h = jnp.tanh(jnp.dot(x, w1_ref[...],
                         preferred_element_type=jnp.float32) + b1_ref[...])
    h = jnp.tanh(jnp.dot(h, w2_ref[...],
                         preferred_element_type=jnp.float32) + b2_ref[...])
    o_ref[...] = jnp.dot(h, w3_ref[...],
                         preferred_element_type=jnp.float32) + b3_ref[...]


def kernel(inputs, table, W1, b1, W2, b2, W3, b3):
    info = plsc.get_sparse_core_info()
    sc_pool = _make_sc_pool(info.num_cores, info.num_subcores)

    ridx = 2 * inputs - jnp.where(inputs >= _SPLIT, 2 * _SPLIT - 1, 0)
    idx2 = ridx.reshape(_B * 2, _CHUNK)
    tab_lin = _transpose_table(table.T).reshape(2 * _SPLIT, _D)
    x = sc_pool(idx2, tab_lin)

    h1 = W1.shape[1]
    w1p = jnp.pad(W1, ((0, 0), (0, 128 - h1)))
    b1p = jnp.pad(b1, (0, 128 - h1)).reshape(1, 128)
    w2p = jnp.pad(W2, ((0, 128 - h1), (0, 128 - h1)))
    b2p = jnp.pad(b2, (0, 128 - h1)).reshape(1, 128)
    w3p = jnp.pad(W3, ((0, 128 - h1), (0, 128 - W3.shape[1])))
    b3p = jnp.pad(b3, (0, 128 - W3.shape[1])).reshape(1, 128)

    logits_pad = pl.pallas_call(
        _mlp_body,
        out_shape=jax.ShapeDtypeStruct((_B, 128), jnp.float32),
    )(x, w1p, b1p, w2p, b2p, w3p, b3p)
    return logits_pad[:, : W3.shape[1]]
